# Initial kernel scaffold; baseline (speedup 1.0000x reference)
import jax
import jax.numpy as jnp
from jax.experimental import pallas as pl


def kernel(x_seq, edge_index, batch, W1, b1, Wp, bp, W2, b2, Wih, Whh, bih, bhh, Wc, bc):
    raise NotImplementedError("write your pallas kernel here")


# speedup vs baseline: 16.9954x; 16.9954x over previous
import functools

import jax
import jax.numpy as jnp
from jax import lax
from jax.experimental import pallas as pl
from jax.experimental.pallas import tpu as pltpu
from jax.experimental.pallas import tpu_sc as plsc

N = 10000
E = 320000
T = 4
B = 8
F_IN = 128
H1 = 16
HID = 128
RATIO = 0.8

NPAD = 10240
NBLK = NPAD // 128
NC = 2
NS = 16
NW = NC * NS
CB = 128
CH = 79
EPT = CH * CB
EPAD = NW * EPT
RPT = NPAD // NS

_HI = jax.lax.Precision.HIGHEST


def _build_seg1(mesh):
    @functools.partial(
        pl.kernel,
        out_type=jax.ShapeDtypeStruct((NW, NPAD), jnp.float32),
        mesh=mesh,
        scratch_types=[
            pltpu.VMEM((NPAD,), jnp.float32),
            pltpu.VMEM((NPAD,), jnp.float32),
            pltpu.VMEM((EPT,), jnp.int32),
            pltpu.VMEM((EPT,), jnp.int32),
        ],
        compiler_params=pltpu.CompilerParams(needs_layout_passes=False),
    )
    def seg1(table_hbm, src_hbm, dst_hbm, zeros_hbm, out_hbm,
             table_v, acc_v, sidx_v, didx_v):
        cid = lax.axis_index("c")
        sid = lax.axis_index("s")
        wid = sid * NC + cid
        pltpu.sync_copy(table_hbm, table_v)
        pltpu.sync_copy(zeros_hbm, acc_v)
        pltpu.sync_copy(src_hbm.at[pl.ds(wid * EPT, EPT)], sidx_v)
        pltpu.sync_copy(dst_hbm.at[pl.ds(wid * EPT, EPT)], didx_v)

        def body(i, carry):
            s16 = sidx_v[pl.ds(i * 16, 16)]
            d16 = didx_v[pl.ds(i * 16, 16)]
            vals = plsc.load_gather(table_v, [s16])
            plsc.addupdate_scatter(acc_v, [d16], vals)
            return carry

        lax.fori_loop(0, EPT // 16, body, 0)
        pltpu.sync_copy(acc_v, out_hbm.at[wid])

    return seg1


def _build_seg16(mesh):
    @functools.partial(
        pl.kernel,
        out_type=jax.ShapeDtypeStruct((NC, NPAD, H1), jnp.float32),
        mesh=mesh,
        scratch_types=[
            pltpu.VMEM((CH, CB), jnp.int32),
            pltpu.VMEM((CH, CB), jnp.int32),
            pltpu.VMEM((CB, H1), jnp.float32),
            pltpu.VMEM_SHARED((NPAD, H1), jnp.float32),
            pltpu.SemaphoreType.DMA,
        ],
        compiler_params=pltpu.CompilerParams(needs_layout_passes=False,
                                             use_tc_tiling_on_sc=False),
    )
    def seg16(table_hbm, src_hbm, dst_hbm, zeros_hbm, out_hbm,
              sidx_v, didx_v, rows_v, acc_sh, sem):
        cid = lax.axis_index("c")
        sid = lax.axis_index("s")
        wid = sid * NC + cid
        rslice = pl.ds(sid * RPT, RPT)
        pltpu.sync_copy(zeros_hbm.at[rslice], acc_sh.at[rslice])
        pltpu.sync_copy(src_hbm.at[wid], sidx_v)
        pltpu.sync_copy(dst_hbm.at[wid], didx_v)
        plsc.subcore_barrier()

        def body(j, carry):
            pltpu.async_copy(table_hbm.at[sidx_v.at[j]], rows_v, sem).wait()
            pltpu.sync_copy(rows_v, acc_sh.at[didx_v.at[j]], add=True)
            return carry

        lax.fori_loop(0, CH, body, 0)
        plsc.subcore_barrier()
        pltpu.sync_copy(acc_sh.at[rslice], out_hbm.at[cid].at[rslice])

    return seg16


def _sum_parts_col(parts):
    ones = jnp.ones((parts.shape[0], 1), jnp.float32)
    return lax.dot_general(parts, ones, (((0,), (0,)), ((), ())),
                           precision=_HI, preferred_element_type=jnp.float32)


def _tc_prep_body(degp_ref, dinv_ref):
    deg = 1.0 + _sum_parts_col(degp_ref[...])
    dinv_ref[...] = jnp.where(deg > 0, lax.rsqrt(deg), 0.0)


def _tc1_body(x_ref, w1_ref, dinv_ref, y1_ref):
    xw = jnp.dot(x_ref[...], w1_ref[...],
                 preferred_element_type=jnp.float32)
    y1_ref[...] = xw * dinv_ref[...]


def _tc2_body(a1_ref, y1_ref, dinv_ref, b1_ref, wp_ref, h_ref, y2_ref):
    agg = a1_ref[0] + a1_ref[1]
    dinv = dinv_ref[...]
    h = jnp.maximum(dinv * agg + y1_ref[...] * dinv + b1_ref[...], 0.0)
    h_ref[...] = h
    s0 = jnp.dot(h, wp_ref[...],
                 preferred_element_type=jnp.float32)
    y2_ref[...] = s0 * dinv


def _tc3_body(sp_ref, y2_ref, dinv_ref, h_ref, bcol_ref, bp_ref,
              bfirst_ref, blast_ref, keep_ref, hp_ref, score_scr, kperf_scr):
    f32 = jnp.float32
    ones_nw = jnp.ones((NW, 1), f32)
    dnums = (((0,), (0,)), ((), ()))
    bv9 = lax.broadcasted_iota(jnp.int32, (1, B + 1), 1).astype(f32)

    def sbody(r, cnt9):
        blk = pl.ds(r * CB, CB)
        aggs_r = lax.dot_general(sp_ref[:, blk], ones_nw, dnums,
                                 precision=_HI, preferred_element_type=f32)
        dinv_r = dinv_ref[blk, :]
        score_scr[blk, :] = dinv_r * aggs_r + y2_ref[blk, :] * dinv_r + bp_ref[0, 0]
        eq = (bcol_ref[blk, :] == bv9).astype(f32)
        return cnt9 + jnp.sum(eq, axis=0, keepdims=True)

    cnt9 = lax.fori_loop(0, NBLK, sbody, jnp.zeros((1, B + 1), f32))
    kper9 = jnp.ceil(f32(RATIO) * cnt9)

    def kbody(r, carry):
        blk = pl.ds(r * CB, CB)
        eq = (bcol_ref[blk, :] == bv9).astype(f32)
        kperf_scr[blk, :] = jnp.sum(eq * kper9, axis=1, keepdims=True)
        return carry

    lax.fori_loop(0, NBLK, kbody, 0)

    ri = lax.broadcasted_iota(jnp.int32, (CB, CB), 0).astype(f32)
    ci = lax.broadcasted_iota(jnp.int32, (CB, CB), 1).astype(f32)
    eyem = (ri == ci).astype(f32)
    onesm = jnp.ones((CB, CB), f32)

    def to_rows(v_col):
        return lax.dot_general(v_col * eyem, onesm, (((1,), (0,)), ((), ())),
                               precision=_HI, preferred_element_type=f32)

    def to_cols(v_col):
        return lax.dot_general(onesm, v_col * eyem, (((1,), (0,)), ((), ())),
                               precision=_HI, preferred_element_type=f32)

    def rbody(r, carry):
        s_r = score_scr[pl.ds(r * CB, CB), :]
        b_r = bcol_ref[pl.ds(r * CB, CB), :]
        k_r = kperf_scr[pl.ds(r * CB, CB), :]
        srow = to_rows(s_r)
        brow = to_rows(b_r)
        irow = ri + f32(CB) * lax.convert_element_type(r, f32)
        bf_r = bfirst_ref[r]
        bl_r = blast_ref[r]

        def cbody(c, rank):
            overlap = (bfirst_ref[c] <= bl_r) & (blast_ref[c] >= bf_r)

            def do(rank):
                s_c = score_scr[pl.ds(c * CB, CB), :]
                b_c = bcol_ref[pl.ds(c * CB, CB), :]
                scol = to_cols(s_c)
                bcolm = to_cols(b_c)
                icol = ci + f32(CB) * lax.convert_element_type(c, f32)
                beats = (scol > srow) | ((scol == srow) & (icol < irow))
                cnt = jnp.where((bcolm == brow) & beats, f32(1), f32(0))
                return rank + jnp.sum(cnt, axis=1, keepdims=True)

            return lax.cond(overlap, do, lambda x: x, rank)

        rank = lax.fori_loop(0, NBLK, cbody, jnp.zeros((CB, 1), f32))
        keep_r = jnp.where(rank < k_r, f32(1), f32(0))
        keep_ref[pl.ds(r * CB, CB), :] = keep_r
        h_r = h_ref[pl.ds(r * CB, CB), :]
        hp_ref[pl.ds(r * CB, CB), :] = h_r * jnp.tanh(s_r) * keep_r
        return carry

    lax.fori_loop(0, NBLK, rbody, 0)


def _tc4_body(dp_ref, keep_ref, hp_ref, dinv2_ref, z_ref):
    indeg2 = _sum_parts_col(dp_ref[...])
    deg2 = keep_ref[...] * (1.0 + indeg2)
    dinv2 = jnp.where(deg2 > 0, lax.rsqrt(deg2), 0.0)
    dinv2_ref[...] = dinv2
    z_ref[...] = hp_ref[...] * dinv2


def _tc5_body(a2_ref, z_ref, dinv2_ref, keep_ref, w2_ref, b2_ref, bcol_ref,
              seq_ref):
    agg2 = a2_ref[0] + a2_ref[1]
    dinv2 = dinv2_ref[...]
    pre = jnp.dot(dinv2 * agg2 + z_ref[...] * dinv2, w2_ref[...],
                  preferred_element_type=jnp.float32)
    h2 = jnp.maximum(keep_ref[...] * (pre + b2_ref[...]), 0.0)
    bv = lax.broadcasted_iota(jnp.int32, (1, B), 1).astype(jnp.float32)
    mask = (bcol_ref[...] == bv).astype(jnp.float32)
    dn = (((0,), (0,)), ((), ()))
    sums = lax.dot_general(mask, h2, dn, precision=_HI,
                           preferred_element_type=jnp.float32)
    cnts = lax.dot_general(mask, keep_ref[...], dn, precision=_HI,
                           preferred_element_type=jnp.float32)
    seq_ref[...] = sums / jnp.maximum(cnts, 1.0)


def _tc6_body(seq_ref, wih_t_ref, whh_t_ref, bias_ref, wc_t_ref, bc_ref,
              out_ref):
    def sigmoid(v):
        return 1.0 / (1.0 + jnp.exp(-v))

    hh = jnp.zeros((B, HID), jnp.float32)
    cc = jnp.zeros((B, HID), jnp.float32)
    for t in range(T):
        g = (jnp.dot(seq_ref[t], wih_t_ref[...],
                     preferred_element_type=jnp.float32)
             + jnp.dot(hh, whh_t_ref[...],
                       preferred_element_type=jnp.float32)
             + bias_ref[...])
        i = sigmoid(g[:, 0 * HID:1 * HID])
        f = sigmoid(g[:, 1 * HID:2 * HID])
        gg = jnp.tanh(g[:, 2 * HID:3 * HID])
        o = sigmoid(g[:, 3 * HID:4 * HID])
        cc = f * cc + i * gg
        hh = o * jnp.tanh(cc)
    out_ref[...] = jnp.dot(hh, wc_t_ref[...],
                           preferred_element_type=jnp.float32) + bc_ref[...]


def _tc_call(body, out_shapes, *args, smem_args=0, scratch_shapes=()):
    n_in = len(args)
    in_specs = [pl.BlockSpec(memory_space=pltpu.VMEM)
                for _ in range(n_in - smem_args)]
    in_specs += [pl.BlockSpec(memory_space=pltpu.SMEM)
                 for _ in range(smem_args)]
    return pl.pallas_call(
        body,
        out_shape=out_shapes,
        in_specs=in_specs,
        out_specs=jax.tree.map(
            lambda _: pl.BlockSpec(memory_space=pltpu.VMEM), out_shapes),
        scratch_shapes=list(scratch_shapes),
    )(*args)


def kernel(x_seq, edge_index, batch, W1, b1, Wp, bp, W2, b2,
           Wih, Whh, bih, bhh, Wc, bc):
    f32 = jnp.float32
    mesh = plsc.VectorSubcoreMesh(core_axis_name="c", subcore_axis_name="s")
    seg1 = _build_seg1(mesh)
    seg16 = _build_seg16(mesh)

    src = edge_index[0].astype(jnp.int32)
    dst = edge_index[1].astype(jnp.int32)
    epad = jnp.full((EPAD - E,), NPAD - 1, jnp.int32)
    src_f = jnp.concatenate([src, epad])
    dst_f = jnp.concatenate([dst, epad])
    src3 = src_f.reshape(NW, CH, CB)
    dst3 = dst_f.reshape(NW, CH, CB)

    batch_p = jnp.concatenate(
        [batch.astype(jnp.int32), jnp.full((NPAD - N,), B, jnp.int32)])
    bcol = batch_p.astype(f32)[:, None]
    bfirst = batch_p[0::CB]
    blast = batch_p[CB - 1::CB]

    zeros1 = jnp.zeros((NPAD,), f32)
    ones1 = jnp.ones((NPAD,), f32)
    zeros16 = jnp.zeros((NPAD, H1), f32)
    xp = jnp.pad(x_seq, ((0, 0), (0, NPAD - N), (0, 0)))

    degp = seg1(ones1, src_f, dst_f, zeros1)
    dinv = _tc_call(_tc_prep_body,
                    jax.ShapeDtypeStruct((NPAD, 1), f32), degp)

    seqs = []
    for t in range(T):
        y1 = _tc_call(_tc1_body, jax.ShapeDtypeStruct((NPAD, H1), f32),
                      xp[t], W1, dinv)
        a1 = seg16(y1, src3, dst3, zeros16)
        h, y2 = _tc_call(
            _tc2_body,
            (jax.ShapeDtypeStruct((NPAD, H1), f32),
             jax.ShapeDtypeStruct((NPAD, 1), f32)),
            a1, y1, dinv, b1.reshape(1, H1), Wp)
        sp = seg1(y2.reshape(NPAD), src_f, dst_f, zeros1)
        keep, hp = _tc_call(
            _tc3_body,
            (jax.ShapeDtypeStruct((NPAD, 1), f32),
             jax.ShapeDtypeStruct((NPAD, H1), f32)),
            sp, y2, dinv, h, bcol, bp.reshape(1, 1), bfirst, blast,
            smem_args=2,
            scratch_shapes=(pltpu.VMEM((NPAD, 1), f32),
                            pltpu.VMEM((NPAD, 1), f32)))
        dp = seg1(keep.reshape(NPAD), src_f, dst_f, zeros1)
        dinv2, z = _tc_call(
            _tc4_body,
            (jax.ShapeDtypeStruct((NPAD, 1), f32),
             jax.ShapeDtypeStruct((NPAD, H1), f32)),
            dp, keep, hp)
        a2 = seg16(z, src3, dst3, zeros16)
        seqs.append(_tc_call(
            _tc5_body, jax.ShapeDtypeStruct((B, HID), f32),
            a2, z, dinv2, keep, W2, b2.reshape(1, HID), bcol))

    seq = jnp.stack(seqs, axis=0)
    out = _tc_call(
        _tc6_body, jax.ShapeDtypeStruct((B, 1), f32),
        seq, Wih.T, Whh.T, (bih + bhh).reshape(1, 4 * HID),
        Wc.T, bc.reshape(1, 1))
    return out


# speedup vs baseline: 21.2043x; 1.2477x over previous
import functools

import jax
import jax.numpy as jnp
from jax import lax
from jax.experimental import pallas as pl
from jax.experimental.pallas import tpu as pltpu
from jax.experimental.pallas import tpu_sc as plsc

N = 10000
E = 320000
T = 4
B = 8
F_IN = 128
H1 = 16
HID = 128
RATIO = 0.8

NPAD = 10240
NBLK = NPAD // 128
BS = 512
NRB = NPAD // BS
NC = 2
NS = 16
NW = NC * NS
CB = 128
CH = 80
EPT = CH * CB
EPAD = NW * EPT
RPT = NPAD // NS
NBUF = 8
TH = T * H1

_HI = jax.lax.Precision.HIGHEST
_SC_PARAMS = pltpu.CompilerParams(needs_layout_passes=False,
                                  use_tc_tiling_on_sc=False)


def _build_deg(mesh):
    @functools.partial(
        pl.kernel,
        out_type=jax.ShapeDtypeStruct((NW, NPAD), jnp.float32),
        mesh=mesh,
        scratch_types=[
            pltpu.VMEM((NPAD,), jnp.float32),
            pltpu.VMEM((EPT,), jnp.int32),
        ],
        compiler_params=_SC_PARAMS,
    )
    def deg(dst_hbm, zeros_hbm, out_hbm, acc_v, didx_v):
        cid = lax.axis_index("c")
        sid = lax.axis_index("s")
        wid = sid * NC + cid
        pltpu.sync_copy(zeros_hbm, acc_v)
        pltpu.sync_copy(dst_hbm.at[pl.ds(wid * EPT, EPT)], didx_v)
        ones16 = jnp.ones((16,), jnp.float32)

        def body(i, carry):
            d16 = didx_v[pl.ds(i * 16, 16)]
            plsc.addupdate_scatter(acc_v, [d16], ones16)
            return carry

        lax.fori_loop(0, EPT // 16, body, 0)
        pltpu.sync_copy(acc_v, out_hbm.at[wid])

    return deg


def _build_seg1(mesh):
    @functools.partial(
        pl.kernel,
        out_type=jax.ShapeDtypeStruct((NW, T, NPAD), jnp.float32),
        mesh=mesh,
        scratch_types=[
            pltpu.VMEM((T, NPAD), jnp.float32),
            pltpu.VMEM((T, NPAD), jnp.float32),
            pltpu.VMEM((EPT,), jnp.int32),
            pltpu.VMEM((EPT,), jnp.int32),
        ],
        compiler_params=_SC_PARAMS,
    )
    def seg1(tables_hbm, src_hbm, dst_hbm, zeros_hbm, out_hbm,
             tables_v, acc_v, sidx_v, didx_v):
        cid = lax.axis_index("c")
        sid = lax.axis_index("s")
        wid = sid * NC + cid
        pltpu.sync_copy(tables_hbm, tables_v)
        pltpu.sync_copy(zeros_hbm, acc_v)
        pltpu.sync_copy(src_hbm.at[pl.ds(wid * EPT, EPT)], sidx_v)
        pltpu.sync_copy(dst_hbm.at[pl.ds(wid * EPT, EPT)], didx_v)

        def body(i, carry):
            s16 = sidx_v[pl.ds(i * 16, 16)]
            d16 = didx_v[pl.ds(i * 16, 16)]
            for tt in range(T):
                t16 = jnp.full((16,), tt, jnp.int32)
                vals = plsc.load_gather(tables_v, [t16, s16])
                plsc.addupdate_scatter(acc_v, [t16, d16], vals)
            return carry

        lax.fori_loop(0, EPT // 16, body, 0)
        pltpu.sync_copy(acc_v, out_hbm.at[wid])

    return seg1


def _build_seg16(mesh):
    @functools.partial(
        pl.kernel,
        out_type=jax.ShapeDtypeStruct((NC, NPAD, TH), jnp.float32),
        mesh=mesh,
        scratch_types=[
            pltpu.VMEM((CH, CB), jnp.int32),
            pltpu.VMEM((CH, CB), jnp.int32),
        ] + [pltpu.VMEM((CB, TH), jnp.float32) for _ in range(NBUF)]
          + [pltpu.VMEM_SHARED((NPAD, TH), jnp.float32)]
          + [pltpu.SemaphoreType.DMA for _ in range(2 * NBUF)],
        compiler_params=_SC_PARAMS,
    )
    def seg16(table_hbm, src_hbm, dst_hbm, zeros_hbm, out_hbm,
              sidx_v, didx_v, *rest):
        rows = rest[:NBUF]
        acc_sh = rest[NBUF]
        gsems = rest[NBUF + 1:NBUF + 1 + NBUF]
        ssems = rest[NBUF + 1 + NBUF:]
        cid = lax.axis_index("c")
        sid = lax.axis_index("s")
        wid = sid * NC + cid
        rslice = pl.ds(sid * RPT, RPT)
        pltpu.sync_copy(zeros_hbm.at[rslice], acc_sh.at[rslice])
        pltpu.sync_copy(src_hbm.at[wid], sidx_v)
        pltpu.sync_copy(dst_hbm.at[wid], didx_v)
        plsc.subcore_barrier()

        def body(i, carry):
            base = i * NBUF
            gd = [pltpu.async_copy(table_hbm.at[sidx_v.at[base + b]],
                                   rows[b], gsems[b])
                  for b in range(NBUF)]
            sd = []
            for b in range(NBUF):
                gd[b].wait()
                sd.append(pltpu.async_copy(
                    rows[b], acc_sh.at[didx_v.at[base + b]], ssems[b],
                    add=True))
            for b in range(NBUF):
                sd[b].wait()
            return carry

        lax.fori_loop(0, CH // NBUF, body, 0)
        plsc.subcore_barrier()
        pltpu.sync_copy(acc_sh.at[rslice], out_hbm.at[cid].at[rslice])

    return seg16


def _sum_parts_col(parts):
    ones = jnp.ones((parts.shape[0], 1), jnp.float32)
    return lax.dot_general(parts, ones, (((0,), (0,)), ((), ())),
                           precision=_HI, preferred_element_type=jnp.float32)


def _place16(x, tt):
    ri = lax.broadcasted_iota(jnp.int32, (H1, TH), 0)
    ci = lax.broadcasted_iota(jnp.int32, (H1, TH), 1)
    e = (ci == ri + tt * H1).astype(jnp.float32)
    return lax.dot_general(x, e, (((1,), (0,)), ((), ())),
                           precision=_HI, preferred_element_type=jnp.float32)


def _col_of(x4, tt):
    e = (lax.broadcasted_iota(jnp.int32, (T, 1), 0) == tt)
    return lax.dot_general(x4, e.astype(jnp.float32),
                           (((1,), (0,)), ((), ())),
                           precision=_HI, preferred_element_type=jnp.float32)


def _place1(x, tt):
    e = (lax.broadcasted_iota(jnp.int32, (1, T), 1) == tt).astype(jnp.float32)
    return lax.dot_general(x, e, (((1,), (0,)), ((), ())),
                           precision=_HI, preferred_element_type=jnp.float32)


def _expand4(x4, width):
    ri = lax.broadcasted_iota(jnp.int32, (T, T * width), 0)
    ci = lax.broadcasted_iota(jnp.int32, (T, T * width), 1)
    e = (ci // width == ri).astype(jnp.float32)
    return lax.dot_general(x4, e, (((1,), (0,)), ((), ())),
                           precision=_HI, preferred_element_type=jnp.float32)


def _eyem():
    ri = lax.broadcasted_iota(jnp.int32, (CB, CB), 0)
    ci = lax.broadcasted_iota(jnp.int32, (CB, CB), 1)
    return (ri == ci).astype(jnp.float32)


def _colify(v_row):
    d = _eyem() * v_row
    return lax.dot_general(d, jnp.ones((CB, 1), jnp.float32),
                           (((1,), (0,)), ((), ())),
                           precision=_HI, preferred_element_type=jnp.float32)


def _rowify(v_col):
    d = v_col * _eyem()
    r = lax.dot_general(jnp.ones((1, CB), jnp.float32), d,
                        (((1,), (0,)), ((), ())),
                        precision=_HI, preferred_element_type=jnp.float32)
    return r.reshape(CB)


def _to_sub(v_row):
    d = _eyem() * v_row
    return lax.dot_general(d, jnp.ones((CB, CB), jnp.float32),
                           (((1,), (0,)), ((), ())),
                           precision=_HI, preferred_element_type=jnp.float32)


def _tc_prep_body(degp_ref, dinv_ref, dinvr_ref):
    def body(rb, carry):
        blk = pl.ds(rb * BS, BS)
        deg = 1.0 + _sum_parts_col(degp_ref[:, blk])
        dinv_ref[blk, :] = jnp.where(deg > 0, lax.rsqrt(deg), 0.0)
        return carry

    lax.fori_loop(0, NRB, body, 0)

    def rbody(rb, carry):
        blk = pl.ds(rb * CB, CB)
        degr = 1.0 + jnp.sum(degp_ref[:, blk], axis=0)
        dinvr_ref[blk] = jnp.where(degr > 0, lax.rsqrt(degr), 0.0)
        return carry

    lax.fori_loop(0, NBLK, rbody, 0)


def _tc1_body(x_ref, w1_ref, dinv_ref, y14_ref):
    def body(rb, carry):
        blk = pl.ds(rb * BS, BS)
        dinv = dinv_ref[blk, :]
        acc = jnp.zeros((BS, TH), jnp.float32)
        for tt in range(T):
            xw = jnp.dot(x_ref[tt, blk, :], w1_ref[...],
                         preferred_element_type=jnp.float32)
            acc = acc + _place16(xw * dinv, tt)
        y14_ref[blk, :] = acc
        return carry

    lax.fori_loop(0, NRB, body, 0)


def _tc2_body(a1_ref, y14_ref, dinv_ref, b14_ref, wp4_ref, h4_ref, y24_ref):
    def body(rb, carry):
        blk = pl.ds(rb * CB, CB)
        dinv = dinv_ref[blk, :]
        agg = a1_ref[0, blk, :] + a1_ref[1, blk, :]
        h4 = jnp.maximum(dinv * agg + y14_ref[blk, :] * dinv + b14_ref[...],
                         0.0)
        h4_ref[blk, :] = h4
        y24 = jnp.dot(h4, wp4_ref[...],
                      preferred_element_type=jnp.float32) * dinv
        for tt in range(T):
            y24_ref[tt, blk] = _rowify(_col_of(y24, tt))
        return carry

    lax.fori_loop(0, NBLK, body, 0)


def _tc3_body(sp_ref, y24_ref, dinvr_ref, h4_ref, brow_ref, bp_ref,
              bfirst_ref, blast_ref, keep4_ref, hp4_ref,
              score_scr, kperf_scr):
    f32 = jnp.float32

    def sbody(rb, cnt9):
        blk = pl.ds(rb * CB, CB)
        dinv = dinvr_ref[blk]
        brow = brow_ref[blk]
        for tt in range(T):
            aggs = jnp.sum(sp_ref[:, tt, blk], axis=0)
            score_scr[tt, blk] = (dinv * aggs + y24_ref[tt, blk] * dinv
                                  + bp_ref[0, 0])
        add = jnp.zeros((1, B + 1), f32)
        for b in range(B + 1):
            sz = jnp.sum(jnp.where(brow == f32(b), f32(1), f32(0)))
            oh = (lax.broadcasted_iota(jnp.int32, (1, B + 1), 1)
                  == b).astype(f32)
            add = add + sz * oh
        return cnt9 + add

    cnt9 = lax.fori_loop(0, NBLK, sbody, jnp.zeros((1, B + 1), f32))
    kper9 = jnp.ceil(f32(RATIO) * cnt9)

    def kbody(rb, carry):
        blk = pl.ds(rb * CB, CB)
        brow = brow_ref[blk]
        kv = jnp.zeros((CB,), f32)
        for b in range(B + 1):
            kv = kv + jnp.where(brow == f32(b), kper9[0, b], f32(0))
        kperf_scr[blk] = kv
        return carry

    lax.fori_loop(0, NBLK, kbody, 0)

    ci = lax.broadcasted_iota(jnp.int32, (CB, CB), 1).astype(f32)
    ris = lax.broadcasted_iota(jnp.int32, (CB, CB), 0).astype(f32)

    for tt in range(T):
        def rbody(r, carry):
            rblk = pl.ds(r * CB, CB)
            s_r = score_scr[tt, rblk]
            b_r = brow_ref[rblk]
            k_r = kperf_scr[rblk]
            irq = ci + f32(CB) * lax.convert_element_type(r, f32)
            bf_r = bfirst_ref[r]
            bl_r = blast_ref[r]

            def cbody(c, rank):
                overlap = (bfirst_ref[c] <= bl_r) & (blast_ref[c] >= bf_r)

                def do(rank):
                    cblk = pl.ds(c * CB, CB)
                    scp = _to_sub(score_scr[tt, cblk])
                    bcp = _to_sub(brow_ref[cblk])
                    icp = ris + f32(CB) * lax.convert_element_type(c, f32)
                    beats = (scp > s_r) | ((scp == s_r) & (icp < irq))
                    cnt = jnp.where((bcp == b_r) & beats, f32(1), f32(0))
                    return rank + jnp.sum(cnt, axis=0)

                return lax.cond(overlap, do, lambda x: x, rank)

            rank = lax.fori_loop(0, NBLK, cbody, jnp.zeros((CB,), f32))
            keep4_ref[tt, rblk] = jnp.where(rank < k_r, f32(1), f32(0))
            return carry

        lax.fori_loop(0, NBLK, rbody, 0)

    def hbody(rb, carry):
        blk = pl.ds(rb * CB, CB)
        acc = jnp.zeros((CB, TH), jnp.float32)
        for tt in range(T):
            tk = jnp.tanh(score_scr[tt, blk]) * keep4_ref[tt, blk]
            h_t = lax.dot_general(
                h4_ref[blk, :], _tsel(tt), (((1,), (0,)), ((), ())),
                precision=_HI, preferred_element_type=jnp.float32)
            acc = acc + _place16(h_t * _colify(tk), tt)
        hp4_ref[blk, :] = acc
        return carry

    lax.fori_loop(0, NBLK, hbody, 0)


def _tsel(tt):
    ri = lax.broadcasted_iota(jnp.int32, (TH, H1), 0)
    ci = lax.broadcasted_iota(jnp.int32, (TH, H1), 1)
    return (ri == ci + tt * H1).astype(jnp.float32)


def _tc4_body(dp_ref, keep4_ref, hp4_ref, dinv24_ref, z4_ref):
    def body(rb, carry):
        blk = pl.ds(rb * CB, CB)
        z4 = jnp.zeros((CB, TH), jnp.float32)
        for tt in range(T):
            indeg2 = jnp.sum(dp_ref[:, tt, blk], axis=0)
            deg2 = keep4_ref[tt, blk] * (1.0 + indeg2)
            dinv2 = jnp.where(deg2 > 0, lax.rsqrt(deg2), 0.0)
            dinv24_ref[tt, blk] = dinv2
            h_t = lax.dot_general(
                hp4_ref[blk, :], _tsel(tt), (((1,), (0,)), ((), ())),
                precision=_HI, preferred_element_type=jnp.float32)
            z4 = z4 + _place16(h_t * _colify(dinv2), tt)
        z4_ref[blk, :] = z4
        return carry

    lax.fori_loop(0, NBLK, body, 0)


def _tc5_body(a2_ref, z4_ref, dinv24_ref, keep4_ref, w2b_ref, b2_ref,
              bcol_ref, wih_t_ref, whh_t_ref, bias_ref, wc_t_ref, bc_ref,
              out_ref):
    f32 = jnp.float32
    bv8 = lax.broadcasted_iota(jnp.int32, (1, B), 1).astype(f32)
    dn = (((0,), (0,)), ((), ()))

    def body(rb, carry):
        sums, cnts = carry
        blk = pl.ds(rb * CB, CB)
        agg2 = a2_ref[0, blk, :] + a2_ref[1, blk, :]
        keep4 = sum(_place1(_colify(keep4_ref[tt, blk]), tt)
                    for tt in range(T))
        dinv24 = sum(_place1(_colify(dinv24_ref[tt, blk]), tt)
                     for tt in range(T))
        dexp = _expand4(dinv24, H1)
        a4 = agg2 * dexp + z4_ref[blk, :] * dexp
        h2all = jnp.dot(a4, w2b_ref[...],
                        preferred_element_type=f32)
        kexp = _expand4(keep4, HID)
        mask = (bcol_ref[blk, :] == bv8).astype(f32)
        new_sums = []
        for tt in range(T):
            h2 = jnp.maximum(
                kexp[:, tt * HID:(tt + 1) * HID]
                * (h2all[:, tt * HID:(tt + 1) * HID] + b2_ref[...]), 0.0)
            new_sums.append(sums[tt] + lax.dot_general(
                mask, h2, dn, precision=_HI, preferred_element_type=f32))
        cnts = cnts + lax.dot_general(mask, keep4, dn, precision=_HI,
                                      preferred_element_type=f32)
        return tuple(new_sums), cnts

    init = (tuple(jnp.zeros((B, HID), f32) for _ in range(T)),
            jnp.zeros((B, T), f32))
    sums, cnts = lax.fori_loop(0, NBLK, body, init)

    def sigmoid(v):
        return 1.0 / (1.0 + jnp.exp(-v))

    hh = jnp.zeros((B, HID), f32)
    cc = jnp.zeros((B, HID), f32)
    for tt in range(T):
        cnt_t = _col_of(cnts, tt)
        seq_t = sums[tt] / jnp.maximum(cnt_t, 1.0)
        g = (jnp.dot(seq_t, wih_t_ref[...], preferred_element_type=f32)
             + jnp.dot(hh, whh_t_ref[...], preferred_element_type=f32)
             + bias_ref[...])
        i = sigmoid(g[:, 0 * HID:1 * HID])
        f = sigmoid(g[:, 1 * HID:2 * HID])
        gg = jnp.tanh(g[:, 2 * HID:3 * HID])
        o = sigmoid(g[:, 3 * HID:4 * HID])
        cc = f * cc + i * gg
        hh = o * jnp.tanh(cc)
    out_ref[...] = jnp.dot(hh, wc_t_ref[...],
                           preferred_element_type=f32) + bc_ref[...]


def _tc_call(body, out_shapes, *args, smem_args=0, scratch_shapes=()):
    n_in = len(args)
    in_specs = [pl.BlockSpec(memory_space=pltpu.VMEM)
                for _ in range(n_in - smem_args)]
    in_specs += [pl.BlockSpec(memory_space=pltpu.SMEM)
                 for _ in range(smem_args)]
    return pl.pallas_call(
        body,
        out_shape=out_shapes,
        in_specs=in_specs,
        out_specs=jax.tree.map(
            lambda _: pl.BlockSpec(memory_space=pltpu.VMEM), out_shapes),
        scratch_shapes=list(scratch_shapes),
    )(*args)


def kernel(x_seq, edge_index, batch, W1, b1, Wp, bp, W2, b2,
           Wih, Whh, bih, bhh, Wc, bc):
    f32 = jnp.float32
    mesh = plsc.VectorSubcoreMesh(core_axis_name="c", subcore_axis_name="s")
    deg_k = _build_deg(mesh)
    seg1 = _build_seg1(mesh)
    seg16 = _build_seg16(mesh)

    src = edge_index[0].astype(jnp.int32)
    dst = edge_index[1].astype(jnp.int32)
    epad = jnp.full((EPAD - E,), NPAD - 1, jnp.int32)
    src_f = jnp.concatenate([src, epad])
    dst_f = jnp.concatenate([dst, epad])
    src3 = src_f.reshape(NW, CH, CB)
    dst3 = dst_f.reshape(NW, CH, CB)

    batch_p = jnp.concatenate(
        [batch.astype(jnp.int32), jnp.full((NPAD - N,), B, jnp.int32)])
    brow = batch_p.astype(f32)
    bcol = brow[:, None]
    bfirst = batch_p[0::CB]
    blast = batch_p[CB - 1::CB]

    zeros1 = jnp.zeros((NPAD,), f32)
    zeros4 = jnp.zeros((T, NPAD), f32)
    zeros64 = jnp.zeros((NPAD, TH), f32)
    xp = jnp.pad(x_seq, ((0, 0), (0, NPAD - N), (0, 0)))

    b14 = jnp.tile(b1, T).reshape(1, TH)
    wp4 = jnp.zeros((TH, T), f32)
    for tt in range(T):
        wp4 = wp4.at[tt * H1:(tt + 1) * H1, tt].set(Wp[:, 0])
    w2b = jnp.zeros((TH, T * HID), f32)
    for tt in range(T):
        w2b = w2b.at[tt * H1:(tt + 1) * H1, tt * HID:(tt + 1) * HID].set(W2)

    degp = deg_k(dst_f, zeros1)
    dinv, dinvr = _tc_call(
        _tc_prep_body,
        (jax.ShapeDtypeStruct((NPAD, 1), f32),
         jax.ShapeDtypeStruct((NPAD,), f32)),
        degp)

    y14 = _tc_call(_tc1_body, jax.ShapeDtypeStruct((NPAD, TH), f32),
                   xp, W1, dinv)
    a1 = seg16(y14, src3, dst3, zeros64)
    h4, y24 = _tc_call(
        _tc2_body,
        (jax.ShapeDtypeStruct((NPAD, TH), f32),
         jax.ShapeDtypeStruct((T, NPAD), f32)),
        a1, y14, dinv, b14, wp4)
    sp = seg1(y24, src_f, dst_f, zeros4)
    keep4, hp4 = _tc_call(
        _tc3_body,
        (jax.ShapeDtypeStruct((T, NPAD), f32),
         jax.ShapeDtypeStruct((NPAD, TH), f32)),
        sp, y24, dinvr, h4, brow, bp.reshape(1, 1), bfirst, blast,
        smem_args=2,
        scratch_shapes=(pltpu.VMEM((T, NPAD), f32),
                        pltpu.VMEM((NPAD,), f32)))
    dp = seg1(keep4, src_f, dst_f, zeros4)
    dinv24, z4 = _tc_call(
        _tc4_body,
        (jax.ShapeDtypeStruct((T, NPAD), f32),
         jax.ShapeDtypeStruct((NPAD, TH), f32)),
        dp, keep4, hp4)
    a2 = seg16(z4, src3, dst3, zeros64)
    out = _tc_call(
        _tc5_body, jax.ShapeDtypeStruct((B, 1), f32),
        a2, z4, dinv24, keep4, w2b, b2.reshape(1, HID), bcol,
        Wih.T, Whh.T, (bih + bhh).reshape(1, 4 * HID),
        Wc.T, bc.reshape(1, 1))
    return out


# speedup vs baseline: 25.6197x; 1.2082x over previous
import functools

import jax
import jax.numpy as jnp
from jax import lax
from jax.experimental import pallas as pl
from jax.experimental.pallas import tpu as pltpu
from jax.experimental.pallas import tpu_sc as plsc

N = 10000
E = 320000
T = 4
B = 8
F_IN = 128
H1 = 16
HID = 128
RATIO = 0.8

NPAD = 10240
NBLK = NPAD // 128
BS = 512
NRB = NPAD // BS
NC = 2
NS = 16
NW = NC * NS
CB = 128
CH = 80
EPT = CH * CB
EPAD = NW * EPT
RPT = NPAD // NS
NBUF = 8
TH = T * H1

_HI = jax.lax.Precision.HIGHEST
_SC_PARAMS = pltpu.CompilerParams(needs_layout_passes=False,
                                  use_tc_tiling_on_sc=False)


def _build_deg(mesh):
    @functools.partial(
        pl.kernel,
        out_type=jax.ShapeDtypeStruct((NW, NPAD), jnp.float32),
        mesh=mesh,
        scratch_types=[
            pltpu.VMEM((NPAD,), jnp.float32),
            pltpu.VMEM((EPT,), jnp.int32),
        ],
        compiler_params=_SC_PARAMS,
    )
    def deg(dst_hbm, zeros_hbm, out_hbm, acc_v, didx_v):
        cid = lax.axis_index("c")
        sid = lax.axis_index("s")
        wid = sid * NC + cid
        pltpu.sync_copy(zeros_hbm, acc_v)
        pltpu.sync_copy(dst_hbm.at[pl.ds(wid * EPT, EPT)], didx_v)
        ones16 = jnp.ones((16,), jnp.float32)

        def body(i, carry):
            d16 = didx_v[pl.ds(i * 16, 16)]
            plsc.addupdate_scatter(acc_v, [d16], ones16)
            return carry

        lax.fori_loop(0, EPT // 16, body, 0)
        pltpu.sync_copy(acc_v, out_hbm.at[wid])

    return deg


def _build_seg1(mesh):
    @functools.partial(
        pl.kernel,
        out_type=jax.ShapeDtypeStruct((NW, T, NPAD), jnp.float32),
        mesh=mesh,
        scratch_types=[
            pltpu.VMEM((T, NPAD), jnp.float32),
            pltpu.VMEM((T, NPAD), jnp.float32),
            pltpu.VMEM((EPT,), jnp.int32),
            pltpu.VMEM((EPT,), jnp.int32),
        ],
        compiler_params=_SC_PARAMS,
    )
    def seg1(tables_hbm, src_hbm, dst_hbm, zeros_hbm, out_hbm,
             tables_v, acc_v, sidx_v, didx_v):
        cid = lax.axis_index("c")
        sid = lax.axis_index("s")
        wid = sid * NC + cid
        pltpu.sync_copy(tables_hbm, tables_v)
        pltpu.sync_copy(zeros_hbm, acc_v)
        pltpu.sync_copy(src_hbm.at[pl.ds(wid * EPT, EPT)], sidx_v)
        pltpu.sync_copy(dst_hbm.at[pl.ds(wid * EPT, EPT)], didx_v)

        def body(i, carry):
            s16 = sidx_v[pl.ds(i * 16, 16)]
            d16 = didx_v[pl.ds(i * 16, 16)]
            for tt in range(T):
                t16 = jnp.full((16,), tt, jnp.int32)
                vals = plsc.load_gather(tables_v, [t16, s16])
                plsc.addupdate_scatter(acc_v, [t16, d16], vals)
            return carry

        lax.fori_loop(0, EPT // 16, body, 0)
        pltpu.sync_copy(acc_v, out_hbm.at[wid])

    return seg1


def _build_seg16(mesh):
    @functools.partial(
        pl.kernel,
        out_type=jax.ShapeDtypeStruct((NC, NPAD, TH), jnp.float32),
        mesh=mesh,
        scratch_types=[
            pltpu.VMEM((CH, CB), jnp.int32),
            pltpu.VMEM((CH, CB), jnp.int32),
        ] + [pltpu.VMEM((CB, TH), jnp.float32) for _ in range(NBUF)]
          + [pltpu.VMEM_SHARED((NPAD, TH), jnp.float32)]
          + [pltpu.SemaphoreType.DMA for _ in range(2 * NBUF)],
        compiler_params=_SC_PARAMS,
    )
    def seg16(table_hbm, src_hbm, dst_hbm, zeros_hbm, out_hbm,
              sidx_v, didx_v, *rest):
        rows = rest[:NBUF]
        acc_sh = rest[NBUF]
        gsems = rest[NBUF + 1:NBUF + 1 + NBUF]
        ssems = rest[NBUF + 1 + NBUF:]
        cid = lax.axis_index("c")
        sid = lax.axis_index("s")
        wid = sid * NC + cid
        rslice = pl.ds(sid * RPT, RPT)
        pltpu.sync_copy(zeros_hbm.at[rslice], acc_sh.at[rslice])
        pltpu.sync_copy(src_hbm.at[wid], sidx_v)
        pltpu.sync_copy(dst_hbm.at[wid], didx_v)
        plsc.subcore_barrier()

        def body(i, carry):
            base = i * NBUF
            gd = [pltpu.async_copy(table_hbm.at[sidx_v.at[base + b]],
                                   rows[b], gsems[b])
                  for b in range(NBUF)]
            sd = []
            for b in range(NBUF):
                gd[b].wait()
                sd.append(pltpu.async_copy(
                    rows[b], acc_sh.at[didx_v.at[base + b]], ssems[b],
                    add=True))
            for b in range(NBUF):
                sd[b].wait()
            return carry

        lax.fori_loop(0, CH // NBUF, body, 0)
        plsc.subcore_barrier()
        pltpu.sync_copy(acc_sh.at[rslice], out_hbm.at[cid].at[rslice])

    return seg16


def _sum_parts_col(parts):
    ones = jnp.ones((parts.shape[0], 1), jnp.float32)
    return lax.dot_general(parts, ones, (((0,), (0,)), ((), ())),
                           precision=_HI, preferred_element_type=jnp.float32)


def _place16(x, tt):
    ri = lax.broadcasted_iota(jnp.int32, (H1, TH), 0)
    ci = lax.broadcasted_iota(jnp.int32, (H1, TH), 1)
    e = (ci == ri + tt * H1).astype(jnp.float32)
    return lax.dot_general(x, e, (((1,), (0,)), ((), ())),
                           precision=_HI, preferred_element_type=jnp.float32)


def _col_of(x4, tt):
    e = (lax.broadcasted_iota(jnp.int32, (T, 1), 0) == tt)
    return lax.dot_general(x4, e.astype(jnp.float32),
                           (((1,), (0,)), ((), ())),
                           precision=_HI, preferred_element_type=jnp.float32)


def _place1(x, tt):
    e = (lax.broadcasted_iota(jnp.int32, (1, T), 1) == tt).astype(jnp.float32)
    return lax.dot_general(x, e, (((1,), (0,)), ((), ())),
                           precision=_HI, preferred_element_type=jnp.float32)


def _expand4(x4, width):
    ri = lax.broadcasted_iota(jnp.int32, (T, T * width), 0)
    ci = lax.broadcasted_iota(jnp.int32, (T, T * width), 1)
    e = (ci // width == ri).astype(jnp.float32)
    return lax.dot_general(x4, e, (((1,), (0,)), ((), ())),
                           precision=_HI, preferred_element_type=jnp.float32)


def _eyem():
    ri = lax.broadcasted_iota(jnp.int32, (CB, CB), 0)
    ci = lax.broadcasted_iota(jnp.int32, (CB, CB), 1)
    return (ri == ci).astype(jnp.float32)


def _colify(v_row):
    d = _eyem() * v_row
    return lax.dot_general(d, jnp.ones((CB, 1), jnp.float32),
                           (((1,), (0,)), ((), ())),
                           precision=_HI, preferred_element_type=jnp.float32)


def _rowify(v_col):
    d = v_col * _eyem()
    r = lax.dot_general(jnp.ones((1, CB), jnp.float32), d,
                        (((1,), (0,)), ((), ())),
                        precision=_HI, preferred_element_type=jnp.float32)
    return r.reshape(CB)


def _tc_prep_body(degp_ref, dinv_ref, dinvr_ref):
    def body(rb, carry):
        blk = pl.ds(rb * BS, BS)
        deg = 1.0 + _sum_parts_col(degp_ref[:, blk])
        dinv_ref[blk, :] = jnp.where(deg > 0, lax.rsqrt(deg), 0.0)
        return carry

    lax.fori_loop(0, NRB, body, 0)

    def rbody(rb, carry):
        blk = pl.ds(rb * CB, CB)
        degr = 1.0 + jnp.sum(degp_ref[:, blk], axis=0)
        dinvr_ref[blk] = jnp.where(degr > 0, lax.rsqrt(degr), 0.0)
        return carry

    lax.fori_loop(0, NBLK, rbody, 0)


def _tc1_body(x_ref, w1_ref, dinv_ref, y14_ref):
    def body(rb, carry):
        blk = pl.ds(rb * BS, BS)
        dinv = dinv_ref[blk, :]
        acc = jnp.zeros((BS, TH), jnp.float32)
        for tt in range(T):
            xw = jnp.dot(x_ref[tt, blk, :], w1_ref[...],
                         preferred_element_type=jnp.float32)
            acc = acc + _place16(xw * dinv, tt)
        y14_ref[blk, :] = acc
        return carry

    lax.fori_loop(0, NRB, body, 0)


def _tc2_body(a1_ref, y14_ref, dinv_ref, b14_ref, wp4_ref, h4_ref, y24_ref):
    def body(rb, carry):
        blk = pl.ds(rb * CB, CB)
        dinv = dinv_ref[blk, :]
        agg = a1_ref[0, blk, :] + a1_ref[1, blk, :]
        h4 = jnp.maximum(dinv * agg + y14_ref[blk, :] * dinv + b14_ref[...],
                         0.0)
        h4_ref[blk, :] = h4
        y24 = jnp.dot(h4, wp4_ref[...],
                      preferred_element_type=jnp.float32) * dinv
        for tt in range(T):
            y24_ref[tt, blk] = _rowify(_col_of(y24, tt))
        return carry

    lax.fori_loop(0, NBLK, body, 0)


def _tc3_body(sp_ref, y24_ref, dinvr_ref, h4_ref, brow_ref, bp_ref,
              clo_ref, chi_ref, keep4_ref, hp4_ref,
              score_scr, kperf_scr):
    f32 = jnp.float32

    def sbody(rb, cnt9):
        blk = pl.ds(rb * CB, CB)
        dinv = dinvr_ref[blk]
        brow = brow_ref[blk]
        for tt in range(T):
            aggs = jnp.sum(sp_ref[:, tt, blk], axis=0)
            score_scr[tt, blk] = (dinv * aggs + y24_ref[tt, blk] * dinv
                                  + bp_ref[0, 0])
        add = jnp.zeros((1, B + 1), f32)
        for b in range(B + 1):
            sz = jnp.sum(jnp.where(brow == f32(b), f32(1), f32(0)))
            oh = (lax.broadcasted_iota(jnp.int32, (1, B + 1), 1)
                  == b).astype(f32)
            add = add + sz * oh
        return cnt9 + add

    cnt9 = lax.fori_loop(0, NBLK, sbody, jnp.zeros((1, B + 1), f32))
    kper9 = jnp.ceil(f32(RATIO) * cnt9)

    def kbody(rb, carry):
        blk = pl.ds(rb * CB, CB)
        brow = brow_ref[blk]
        kv = jnp.zeros((CB,), f32)
        for b in range(B + 1):
            kv = kv + jnp.where(brow == f32(b), kper9[0, b], f32(0))
        kperf_scr[blk] = kv
        return carry

    lax.fori_loop(0, NBLK, kbody, 0)

    icol0 = lax.broadcasted_iota(jnp.int32, (CB,), 0).astype(f32)
    irow0 = lax.broadcasted_iota(jnp.int32, (CB, 1), 0).astype(f32)

    for tt in range(T):
        def rbody(r, carry):
            rblk = pl.ds(r * CB, CB)
            s_rc = _colify(score_scr[tt, rblk])
            b_rc = _colify(brow_ref[rblk])
            k_rc = _colify(kperf_scr[rblk])
            irow = irow0 + f32(CB) * lax.convert_element_type(r, f32)

            def cbody(c, rank):
                cblk = pl.ds(c * CB, CB)
                s_c = score_scr[tt, cblk]
                b_c = brow_ref[cblk]
                icol = icol0 + f32(CB) * lax.convert_element_type(c, f32)
                beats = (s_c > s_rc) | ((s_c == s_rc) & (icol < irow))
                cnt = jnp.where((b_c == b_rc) & beats, f32(1), f32(0))
                return rank + jnp.sum(cnt, axis=1, keepdims=True)

            rank = lax.fori_loop(clo_ref[r], chi_ref[r], cbody,
                                 jnp.zeros((CB, 1), f32))
            keep4_ref[tt, rblk] = _rowify(
                jnp.where(rank < k_rc, f32(1), f32(0)))
            return carry

        lax.fori_loop(0, NBLK, rbody, 0)

    def hbody(rb, carry):
        blk = pl.ds(rb * CB, CB)
        acc = jnp.zeros((CB, TH), jnp.float32)
        for tt in range(T):
            tk = jnp.tanh(score_scr[tt, blk]) * keep4_ref[tt, blk]
            h_t = lax.dot_general(
                h4_ref[blk, :], _tsel(tt), (((1,), (0,)), ((), ())),
                precision=_HI, preferred_element_type=jnp.float32)
            acc = acc + _place16(h_t * _colify(tk), tt)
        hp4_ref[blk, :] = acc
        return carry

    lax.fori_loop(0, NBLK, hbody, 0)


def _tsel(tt):
    ri = lax.broadcasted_iota(jnp.int32, (TH, H1), 0)
    ci = lax.broadcasted_iota(jnp.int32, (TH, H1), 1)
    return (ri == ci + tt * H1).astype(jnp.float32)


def _tc4_body(dp_ref, keep4_ref, hp4_ref, dinv24_ref, z4_ref):
    def body(rb, carry):
        blk = pl.ds(rb * CB, CB)
        z4 = jnp.zeros((CB, TH), jnp.float32)
        for tt in range(T):
            indeg2 = jnp.sum(dp_ref[:, tt, blk], axis=0)
            deg2 = keep4_ref[tt, blk] * (1.0 + indeg2)
            dinv2 = jnp.where(deg2 > 0, lax.rsqrt(deg2), 0.0)
            dinv24_ref[tt, blk] = dinv2
            h_t = lax.dot_general(
                hp4_ref[blk, :], _tsel(tt), (((1,), (0,)), ((), ())),
                precision=_HI, preferred_element_type=jnp.float32)
            z4 = z4 + _place16(h_t * _colify(dinv2), tt)
        z4_ref[blk, :] = z4
        return carry

    lax.fori_loop(0, NBLK, body, 0)


def _tc5_body(a2_ref, z4_ref, dinv24_ref, keep4_ref, w2b_ref, b2_ref,
              bcol_ref, wih_t_ref, whh_t_ref, bias_ref, wc_t_ref, bc_ref,
              out_ref):
    f32 = jnp.float32
    bv8 = lax.broadcasted_iota(jnp.int32, (1, B), 1).astype(f32)
    dn = (((0,), (0,)), ((), ()))

    def body(rb, carry):
        sums, cnts = carry
        blk = pl.ds(rb * CB, CB)
        agg2 = a2_ref[0, blk, :] + a2_ref[1, blk, :]
        keep4 = sum(_place1(_colify(keep4_ref[tt, blk]), tt)
                    for tt in range(T))
        dinv24 = sum(_place1(_colify(dinv24_ref[tt, blk]), tt)
                     for tt in range(T))
        dexp = _expand4(dinv24, H1)
        a4 = agg2 * dexp + z4_ref[blk, :] * dexp
        h2all = jnp.dot(a4, w2b_ref[...],
                        preferred_element_type=f32)
        kexp = _expand4(keep4, HID)
        mask = (bcol_ref[blk, :] == bv8).astype(f32)
        new_sums = []
        for tt in range(T):
            h2 = jnp.maximum(
                kexp[:, tt * HID:(tt + 1) * HID]
                * (h2all[:, tt * HID:(tt + 1) * HID] + b2_ref[...]), 0.0)
            new_sums.append(sums[tt] + lax.dot_general(
                mask, h2, dn, precision=_HI, preferred_element_type=f32))
        cnts = cnts + lax.dot_general(mask, keep4, dn, precision=_HI,
                                      preferred_element_type=f32)
        return tuple(new_sums), cnts

    init = (tuple(jnp.zeros((B, HID), f32) for _ in range(T)),
            jnp.zeros((B, T), f32))
    sums, cnts = lax.fori_loop(0, NBLK, body, init)

    def sigmoid(v):
        return 1.0 / (1.0 + jnp.exp(-v))

    hh = jnp.zeros((B, HID), f32)
    cc = jnp.zeros((B, HID), f32)
    for tt in range(T):
        cnt_t = _col_of(cnts, tt)
        seq_t = sums[tt] / jnp.maximum(cnt_t, 1.0)
        g = (jnp.dot(seq_t, wih_t_ref[...], preferred_element_type=f32)
             + jnp.dot(hh, whh_t_ref[...], preferred_element_type=f32)
             + bias_ref[...])
        i = sigmoid(g[:, 0 * HID:1 * HID])
        f = sigmoid(g[:, 1 * HID:2 * HID])
        gg = jnp.tanh(g[:, 2 * HID:3 * HID])
        o = sigmoid(g[:, 3 * HID:4 * HID])
        cc = f * cc + i * gg
        hh = o * jnp.tanh(cc)
    out_ref[...] = jnp.dot(hh, wc_t_ref[...],
                           preferred_element_type=f32) + bc_ref[...]


def _tc_call(body, out_shapes, *args, smem_args=0, scratch_shapes=()):
    n_in = len(args)
    in_specs = [pl.BlockSpec(memory_space=pltpu.VMEM)
                for _ in range(n_in - smem_args)]
    in_specs += [pl.BlockSpec(memory_space=pltpu.SMEM)
                 for _ in range(smem_args)]
    return pl.pallas_call(
        body,
        out_shape=out_shapes,
        in_specs=in_specs,
        out_specs=jax.tree.map(
            lambda _: pl.BlockSpec(memory_space=pltpu.VMEM), out_shapes),
        scratch_shapes=list(scratch_shapes),
    )(*args)


def kernel(x_seq, edge_index, batch, W1, b1, Wp, bp, W2, b2,
           Wih, Whh, bih, bhh, Wc, bc):
    f32 = jnp.float32
    mesh = plsc.VectorSubcoreMesh(core_axis_name="c", subcore_axis_name="s")
    deg_k = _build_deg(mesh)
    seg1 = _build_seg1(mesh)
    seg16 = _build_seg16(mesh)

    src = edge_index[0].astype(jnp.int32)
    dst = edge_index[1].astype(jnp.int32)
    epad = jnp.full((EPAD - E,), NPAD - 1, jnp.int32)
    src_f = jnp.concatenate([src, epad])
    dst_f = jnp.concatenate([dst, epad])
    src3 = src_f.reshape(NW, CH, CB)
    dst3 = dst_f.reshape(NW, CH, CB)

    batch_p = jnp.concatenate(
        [batch.astype(jnp.int32), jnp.full((NPAD - N,), B, jnp.int32)])
    brow = batch_p.astype(f32)
    bcol = brow[:, None]
    bfirst = batch_p[0::CB]
    blast = batch_p[CB - 1::CB]
    c_lo = jnp.searchsorted(blast, bfirst, side="left").astype(jnp.int32)
    c_hi = jnp.searchsorted(bfirst, blast, side="right").astype(jnp.int32)

    zeros1 = jnp.zeros((NPAD,), f32)
    zeros4 = jnp.zeros((T, NPAD), f32)
    zeros64 = jnp.zeros((NPAD, TH), f32)
    xp = jnp.pad(x_seq, ((0, 0), (0, NPAD - N), (0, 0)))

    b14 = jnp.tile(b1, T).reshape(1, TH)
    wp4 = jnp.zeros((TH, T), f32)
    for tt in range(T):
        wp4 = wp4.at[tt * H1:(tt + 1) * H1, tt].set(Wp[:, 0])
    w2b = jnp.zeros((TH, T * HID), f32)
    for tt in range(T):
        w2b = w2b.at[tt * H1:(tt + 1) * H1, tt * HID:(tt + 1) * HID].set(W2)

    degp = deg_k(dst_f, zeros1)
    dinv, dinvr = _tc_call(
        _tc_prep_body,
        (jax.ShapeDtypeStruct((NPAD, 1), f32),
         jax.ShapeDtypeStruct((NPAD,), f32)),
        degp)

    y14 = _tc_call(_tc1_body, jax.ShapeDtypeStruct((NPAD, TH), f32),
                   xp, W1, dinv)
    a1 = seg16(y14, src3, dst3, zeros64)
    h4, y24 = _tc_call(
        _tc2_body,
        (jax.ShapeDtypeStruct((NPAD, TH), f32),
         jax.ShapeDtypeStruct((T, NPAD), f32)),
        a1, y14, dinv, b14, wp4)
    sp = seg1(y24, src_f, dst_f, zeros4)
    keep4, hp4 = _tc_call(
        _tc3_body,
        (jax.ShapeDtypeStruct((T, NPAD), f32),
         jax.ShapeDtypeStruct((NPAD, TH), f32)),
        sp, y24, dinvr, h4, brow, bp.reshape(1, 1), c_lo, c_hi,
        smem_args=2,
        scratch_shapes=(pltpu.VMEM((T, NPAD), f32),
                        pltpu.VMEM((NPAD,), f32)))
    dp = seg1(keep4, src_f, dst_f, zeros4)
    dinv24, z4 = _tc_call(
        _tc4_body,
        (jax.ShapeDtypeStruct((T, NPAD), f32),
         jax.ShapeDtypeStruct((NPAD, TH), f32)),
        dp, keep4, hp4)
    a2 = seg16(z4, src3, dst3, zeros64)
    out = _tc_call(
        _tc5_body, jax.ShapeDtypeStruct((B, 1), f32),
        a2, z4, dinv24, keep4, w2b, b2.reshape(1, HID), bcol,
        Wih.T, Whh.T, (bih + bhh).reshape(1, 4 * HID),
        Wc.T, bc.reshape(1, 1))
    return out


# speedup vs baseline: 28.5388x; 1.1139x over previous
import functools

import jax
import jax.numpy as jnp
from jax import lax
from jax.experimental import pallas as pl
from jax.experimental.pallas import tpu as pltpu
from jax.experimental.pallas import tpu_sc as plsc

N = 10000
E = 320000
T = 4
B = 8
F_IN = 128
H1 = 16
HID = 128
RATIO = 0.8

NPAD = 10240
NBLK = NPAD // 128
BS = 512
NRB = NPAD // BS
NC = 2
NS = 16
NW = NC * NS
CB = 128
CH = 80
EPT = CH * CB
EPAD = NW * EPT
RPT = NPAD // NS
NBUF = 8
TH = T * H1

_HI = jax.lax.Precision.HIGHEST
_SC_PARAMS = pltpu.CompilerParams(needs_layout_passes=False,
                                  use_tc_tiling_on_sc=False)


def _build_deg(mesh):
    @functools.partial(
        pl.kernel,
        out_type=jax.ShapeDtypeStruct((NW, NPAD), jnp.float32),
        mesh=mesh,
        scratch_types=[
            pltpu.VMEM((NPAD,), jnp.float32),
            pltpu.VMEM((EPT,), jnp.int32),
        ],
        compiler_params=_SC_PARAMS,
    )
    def deg(dst_hbm, zeros_hbm, out_hbm, acc_v, didx_v):
        cid = lax.axis_index("c")
        sid = lax.axis_index("s")
        wid = sid * NC + cid
        pltpu.sync_copy(zeros_hbm, acc_v)
        pltpu.sync_copy(dst_hbm.at[pl.ds(wid * EPT, EPT)], didx_v)
        ones16 = jnp.ones((16,), jnp.float32)

        def body(i, carry):
            d16 = didx_v[pl.ds(i * 16, 16)]
            plsc.addupdate_scatter(acc_v, [d16], ones16)
            return carry

        lax.fori_loop(0, EPT // 16, body, 0)
        pltpu.sync_copy(acc_v, out_hbm.at[wid])

    return deg


def _build_seg1(mesh):
    @functools.partial(
        pl.kernel,
        out_type=jax.ShapeDtypeStruct((NW, T, NPAD), jnp.float32),
        mesh=mesh,
        scratch_types=[
            pltpu.VMEM((T, NPAD), jnp.float32),
            pltpu.VMEM((T, NPAD), jnp.float32),
            pltpu.VMEM((EPT,), jnp.int32),
            pltpu.VMEM((EPT,), jnp.int32),
        ],
        compiler_params=_SC_PARAMS,
    )
    def seg1(tables_hbm, src_hbm, dst_hbm, zeros_hbm, out_hbm,
             tables_v, acc_v, sidx_v, didx_v):
        cid = lax.axis_index("c")
        sid = lax.axis_index("s")
        wid = sid * NC + cid
        pltpu.sync_copy(tables_hbm, tables_v)
        pltpu.sync_copy(zeros_hbm, acc_v)
        pltpu.sync_copy(src_hbm.at[pl.ds(wid * EPT, EPT)], sidx_v)
        pltpu.sync_copy(dst_hbm.at[pl.ds(wid * EPT, EPT)], didx_v)

        def body(i, carry):
            s16 = sidx_v[pl.ds(i * 16, 16)]
            d16 = didx_v[pl.ds(i * 16, 16)]
            for tt in range(T):
                t16 = jnp.full((16,), tt, jnp.int32)
                vals = plsc.load_gather(tables_v, [t16, s16])
                plsc.addupdate_scatter(acc_v, [t16, d16], vals)
            return carry

        lax.fori_loop(0, EPT // 16, body, 0)
        pltpu.sync_copy(acc_v, out_hbm.at[wid])

    return seg1


def _build_seg16(mesh):
    @functools.partial(
        pl.kernel,
        out_type=jax.ShapeDtypeStruct((NC, NPAD, TH), jnp.float32),
        mesh=mesh,
        scratch_types=[
            pltpu.VMEM((CH, CB), jnp.int32),
            pltpu.VMEM((CH, CB), jnp.int32),
        ] + [pltpu.VMEM((CB, TH), jnp.float32) for _ in range(NBUF)]
          + [pltpu.VMEM_SHARED((NPAD, TH), jnp.float32)]
          + [pltpu.SemaphoreType.DMA for _ in range(2 * NBUF)],
        compiler_params=_SC_PARAMS,
    )
    def seg16(table_hbm, src_hbm, dst_hbm, zeros_hbm, out_hbm,
              sidx_v, didx_v, *rest):
        rows = rest[:NBUF]
        acc_sh = rest[NBUF]
        gsems = rest[NBUF + 1:NBUF + 1 + NBUF]
        ssems = rest[NBUF + 1 + NBUF:]
        cid = lax.axis_index("c")
        sid = lax.axis_index("s")
        wid = sid * NC + cid
        rslice = pl.ds(sid * RPT, RPT)
        pltpu.sync_copy(zeros_hbm.at[rslice], acc_sh.at[rslice])
        pltpu.sync_copy(src_hbm.at[wid], sidx_v)
        pltpu.sync_copy(dst_hbm.at[wid], didx_v)
        plsc.subcore_barrier()

        def body(i, carry):
            base = i * NBUF
            gd = [pltpu.async_copy(table_hbm.at[sidx_v.at[base + b]],
                                   rows[b], gsems[b])
                  for b in range(NBUF)]
            sd = []
            for b in range(NBUF):
                gd[b].wait()
                sd.append(pltpu.async_copy(
                    rows[b], acc_sh.at[didx_v.at[base + b]], ssems[b],
                    add=True))
            for b in range(NBUF):
                sd[b].wait()
            return carry

        lax.fori_loop(0, CH // NBUF, body, 0)
        plsc.subcore_barrier()
        pltpu.sync_copy(acc_sh.at[rslice], out_hbm.at[cid].at[rslice])

    return seg16


def _sum_parts_col(parts):
    ones = jnp.ones((parts.shape[0], 1), jnp.float32)
    return lax.dot_general(parts, ones, (((0,), (0,)), ((), ())),
                           precision=_HI, preferred_element_type=jnp.float32)


def _place16(x, tt):
    ri = lax.broadcasted_iota(jnp.int32, (H1, TH), 0)
    ci = lax.broadcasted_iota(jnp.int32, (H1, TH), 1)
    e = (ci == ri + tt * H1).astype(jnp.float32)
    return lax.dot_general(x, e, (((1,), (0,)), ((), ())),
                           precision=_HI, preferred_element_type=jnp.float32)


def _col_of(x4, tt):
    e = (lax.broadcasted_iota(jnp.int32, (T, 1), 0) == tt)
    return lax.dot_general(x4, e.astype(jnp.float32),
                           (((1,), (0,)), ((), ())),
                           precision=_HI, preferred_element_type=jnp.float32)


def _place1(x, tt):
    e = (lax.broadcasted_iota(jnp.int32, (1, T), 1) == tt).astype(jnp.float32)
    return lax.dot_general(x, e, (((1,), (0,)), ((), ())),
                           precision=_HI, preferred_element_type=jnp.float32)


def _expand4(x4, width):
    ri = lax.broadcasted_iota(jnp.int32, (T, T * width), 0)
    ci = lax.broadcasted_iota(jnp.int32, (T, T * width), 1)
    e = (ci // width == ri).astype(jnp.float32)
    return lax.dot_general(x4, e, (((1,), (0,)), ((), ())),
                           precision=_HI, preferred_element_type=jnp.float32)


def _eyem():
    ri = lax.broadcasted_iota(jnp.int32, (CB, CB), 0)
    ci = lax.broadcasted_iota(jnp.int32, (CB, CB), 1)
    return (ri == ci).astype(jnp.float32)


def _colify(v_row):
    d = _eyem() * v_row
    return lax.dot_general(d, jnp.ones((CB, 1), jnp.float32),
                           (((1,), (0,)), ((), ())),
                           precision=_HI, preferred_element_type=jnp.float32)


def _rowify(v_col):
    d = v_col * _eyem()
    r = lax.dot_general(jnp.ones((1, CB), jnp.float32), d,
                        (((1,), (0,)), ((), ())),
                        precision=_HI, preferred_element_type=jnp.float32)
    return r.reshape(CB)


def _tc_prep_body(degp_ref, dinv_ref, dinvr_ref):
    def body(rb, carry):
        blk = pl.ds(rb * BS, BS)
        deg = 1.0 + _sum_parts_col(degp_ref[:, blk])
        dinv_ref[blk, :] = jnp.where(deg > 0, lax.rsqrt(deg), 0.0)
        return carry

    lax.fori_loop(0, NRB, body, 0)

    def rbody(rb, carry):
        blk = pl.ds(rb * CB, CB)
        degr = 1.0 + jnp.sum(degp_ref[:, blk], axis=0)
        dinvr_ref[blk] = jnp.where(degr > 0, lax.rsqrt(degr), 0.0)
        return carry

    lax.fori_loop(0, NBLK, rbody, 0)


def _tc1_body(x_ref, w1_ref, dinv_ref, y14_ref):
    def body(rb, carry):
        blk = pl.ds(rb * BS, BS)
        dinv = dinv_ref[blk, :]
        acc = jnp.zeros((BS, TH), jnp.float32)
        for tt in range(T):
            xw = jnp.dot(x_ref[tt, blk, :], w1_ref[...],
                         preferred_element_type=jnp.float32)
            acc = acc + _place16(xw * dinv, tt)
        y14_ref[blk, :] = acc
        return carry

    lax.fori_loop(0, NRB, body, 0)


def _tc2_body(a1_ref, y14_ref, dinv_ref, b14_ref, wp4_ref, h4_ref, y24_ref):
    def body(rb, carry):
        blk = pl.ds(rb * CB, CB)
        dinv = dinv_ref[blk, :]
        agg = a1_ref[0, blk, :] + a1_ref[1, blk, :]
        h4 = jnp.maximum(dinv * agg + y14_ref[blk, :] * dinv + b14_ref[...],
                         0.0)
        h4_ref[blk, :] = h4
        y24 = jnp.dot(h4, wp4_ref[...],
                      preferred_element_type=jnp.float32) * dinv
        for tt in range(T):
            y24_ref[tt, blk] = _rowify(_col_of(y24, tt))
        return carry

    lax.fori_loop(0, NBLK, body, 0)


def _tc3_body(sp_ref, y24_ref, dinvr_ref, h4_ref, brow_ref, bp_ref,
              clo_ref, chi_ref, keep4_ref, hp4_ref,
              score_scr, kperf_scr):
    f32 = jnp.float32

    def sbody(rb, cnt9):
        blk = pl.ds(rb * CB, CB)
        dinv = dinvr_ref[blk]
        brow = brow_ref[blk]
        for tt in range(T):
            aggs = jnp.sum(sp_ref[:, tt, blk], axis=0)
            score_scr[tt, blk] = (dinv * aggs + y24_ref[tt, blk] * dinv
                                  + bp_ref[0, 0])
        add = jnp.zeros((1, B + 1), f32)
        for b in range(B + 1):
            sz = jnp.sum(jnp.where(brow == f32(b), f32(1), f32(0)))
            oh = (lax.broadcasted_iota(jnp.int32, (1, B + 1), 1)
                  == b).astype(f32)
            add = add + sz * oh
        return cnt9 + add

    cnt9 = lax.fori_loop(0, NBLK, sbody, jnp.zeros((1, B + 1), f32))
    kper9 = jnp.ceil(f32(RATIO) * cnt9)

    def kbody(rb, carry):
        blk = pl.ds(rb * CB, CB)
        brow = brow_ref[blk]
        kv = jnp.zeros((CB,), f32)
        for b in range(B + 1):
            kv = kv + jnp.where(brow == f32(b), kper9[0, b], f32(0))
        kperf_scr[blk] = kv
        return carry

    lax.fori_loop(0, NBLK, kbody, 0)

    icol0 = lax.broadcasted_iota(jnp.int32, (CB,), 0).astype(f32)
    irow0 = lax.broadcasted_iota(jnp.int32, (CB, 1), 0).astype(f32)

    def rbody(r, carry):
        rblk = pl.ds(r * CB, CB)
        b_rc = _colify(brow_ref[rblk])
        k_rc = _colify(kperf_scr[rblk])
        s_rc = [_colify(score_scr[tt, rblk]) for tt in range(T)]
        irow = irow0 + f32(CB) * lax.convert_element_type(r, f32)

        def cbody(c, ranks):
            cblk = pl.ds(c * CB, CB)
            b_c = brow_ref[cblk]
            icol = icol0 + f32(CB) * lax.convert_element_type(c, f32)
            same = b_c == b_rc
            tie_lt = icol < irow
            out = []
            for tt in range(T):
                s_c = score_scr[tt, cblk]
                beats = (s_c > s_rc[tt]) | ((s_c == s_rc[tt]) & tie_lt)
                cnt = jnp.where(same & beats, f32(1), f32(0))
                out.append(ranks[tt] + jnp.sum(cnt, axis=1, keepdims=True))
            return tuple(out)

        ranks = lax.fori_loop(clo_ref[r], chi_ref[r], cbody,
                              tuple(jnp.zeros((CB, 1), f32)
                                    for _ in range(T)))
        for tt in range(T):
            keep4_ref[tt, rblk] = _rowify(
                jnp.where(ranks[tt] < k_rc, f32(1), f32(0)))
        return carry

    lax.fori_loop(0, NBLK, rbody, 0)

    def hbody(rb, carry):
        blk = pl.ds(rb * CB, CB)
        acc = jnp.zeros((CB, TH), jnp.float32)
        for tt in range(T):
            tk = jnp.tanh(score_scr[tt, blk]) * keep4_ref[tt, blk]
            h_t = lax.dot_general(
                h4_ref[blk, :], _tsel(tt), (((1,), (0,)), ((), ())),
                precision=_HI, preferred_element_type=jnp.float32)
            acc = acc + _place16(h_t * _colify(tk), tt)
        hp4_ref[blk, :] = acc
        return carry

    lax.fori_loop(0, NBLK, hbody, 0)


def _tsel(tt):
    ri = lax.broadcasted_iota(jnp.int32, (TH, H1), 0)
    ci = lax.broadcasted_iota(jnp.int32, (TH, H1), 1)
    return (ri == ci + tt * H1).astype(jnp.float32)


def _tc4_body(dp_ref, keep4_ref, hp4_ref, dinv24_ref, z4_ref):
    def body(rb, carry):
        blk = pl.ds(rb * CB, CB)
        z4 = jnp.zeros((CB, TH), jnp.float32)
        for tt in range(T):
            indeg2 = jnp.sum(dp_ref[:, tt, blk], axis=0)
            deg2 = keep4_ref[tt, blk] * (1.0 + indeg2)
            dinv2 = jnp.where(deg2 > 0, lax.rsqrt(deg2), 0.0)
            dinv24_ref[tt, blk] = dinv2
            h_t = lax.dot_general(
                hp4_ref[blk, :], _tsel(tt), (((1,), (0,)), ((), ())),
                precision=_HI, preferred_element_type=jnp.float32)
            z4 = z4 + _place16(h_t * _colify(dinv2), tt)
        z4_ref[blk, :] = z4
        return carry

    lax.fori_loop(0, NBLK, body, 0)


def _tc5_body(a2_ref, z4_ref, dinv24_ref, keep4_ref, w2b_ref, b2_ref,
              bcol_ref, wih_t_ref, whh_t_ref, bias_ref, wc_t_ref, bc_ref,
              out_ref):
    f32 = jnp.float32
    bv8 = lax.broadcasted_iota(jnp.int32, (1, B), 1).astype(f32)
    dn = (((0,), (0,)), ((), ()))

    def body(rb, carry):
        sums, cnts = carry
        blk = pl.ds(rb * CB, CB)
        raw = jnp.dot(a2_ref[0, blk, :] + a2_ref[1, blk, :] + z4_ref[blk, :],
                      w2b_ref[...], preferred_element_type=f32)
        mask = (bcol_ref[blk, :] == bv8).astype(f32)
        new_sums = []
        keep_cols = []
        for tt in range(T):
            keep_c = _colify(keep4_ref[tt, blk])
            keep_cols.append(keep_c)
            dinv2_c = _colify(dinv24_ref[tt, blk])
            h2 = jnp.maximum(
                keep_c * (dinv2_c * raw[:, tt * HID:(tt + 1) * HID]
                          + b2_ref[...]), 0.0)
            new_sums.append(sums[tt] + lax.dot_general(
                mask, h2, dn, precision=_HI, preferred_element_type=f32))
        keep4 = sum(_place1(keep_cols[tt], tt) for tt in range(T))
        cnts = cnts + lax.dot_general(mask, keep4, dn, precision=_HI,
                                      preferred_element_type=f32)
        return tuple(new_sums), cnts

    init = (tuple(jnp.zeros((B, HID), f32) for _ in range(T)),
            jnp.zeros((B, T), f32))
    sums, cnts = lax.fori_loop(0, NBLK, body, init)

    def sigmoid(v):
        return 1.0 / (1.0 + jnp.exp(-v))

    hh = jnp.zeros((B, HID), f32)
    cc = jnp.zeros((B, HID), f32)
    for tt in range(T):
        cnt_t = _col_of(cnts, tt)
        seq_t = sums[tt] / jnp.maximum(cnt_t, 1.0)
        g = (jnp.dot(seq_t, wih_t_ref[...], preferred_element_type=f32)
             + jnp.dot(hh, whh_t_ref[...], preferred_element_type=f32)
             + bias_ref[...])
        i = sigmoid(g[:, 0 * HID:1 * HID])
        f = sigmoid(g[:, 1 * HID:2 * HID])
        gg = jnp.tanh(g[:, 2 * HID:3 * HID])
        o = sigmoid(g[:, 3 * HID:4 * HID])
        cc = f * cc + i * gg
        hh = o * jnp.tanh(cc)
    out_ref[...] = jnp.dot(hh, wc_t_ref[...],
                           preferred_element_type=f32) + bc_ref[...]


def _tc_call(body, out_shapes, *args, smem_args=0, scratch_shapes=()):
    n_in = len(args)
    in_specs = [pl.BlockSpec(memory_space=pltpu.VMEM)
                for _ in range(n_in - smem_args)]
    in_specs += [pl.BlockSpec(memory_space=pltpu.SMEM)
                 for _ in range(smem_args)]
    return pl.pallas_call(
        body,
        out_shape=out_shapes,
        in_specs=in_specs,
        out_specs=jax.tree.map(
            lambda _: pl.BlockSpec(memory_space=pltpu.VMEM), out_shapes),
        scratch_shapes=list(scratch_shapes),
    )(*args)


def kernel(x_seq, edge_index, batch, W1, b1, Wp, bp, W2, b2,
           Wih, Whh, bih, bhh, Wc, bc):
    f32 = jnp.float32
    mesh = plsc.VectorSubcoreMesh(core_axis_name="c", subcore_axis_name="s")
    deg_k = _build_deg(mesh)
    seg1 = _build_seg1(mesh)
    seg16 = _build_seg16(mesh)

    src = edge_index[0].astype(jnp.int32)
    dst = edge_index[1].astype(jnp.int32)
    epad = jnp.full((EPAD - E,), NPAD - 1, jnp.int32)
    src_f = jnp.concatenate([src, epad])
    dst_f = jnp.concatenate([dst, epad])
    src3 = src_f.reshape(NW, CH, CB)
    dst3 = dst_f.reshape(NW, CH, CB)

    batch_p = jnp.concatenate(
        [batch.astype(jnp.int32), jnp.full((NPAD - N,), B, jnp.int32)])
    brow = batch_p.astype(f32)
    bcol = brow[:, None]
    bfirst = batch_p[0::CB]
    blast = batch_p[CB - 1::CB]
    c_lo = jnp.searchsorted(blast, bfirst, side="left").astype(jnp.int32)
    c_hi = jnp.searchsorted(bfirst, blast, side="right").astype(jnp.int32)

    zeros1 = jnp.zeros((NPAD,), f32)
    zeros4 = jnp.zeros((T, NPAD), f32)
    zeros64 = jnp.zeros((NPAD, TH), f32)
    xp = jnp.pad(x_seq, ((0, 0), (0, NPAD - N), (0, 0)))

    b14 = jnp.tile(b1, T).reshape(1, TH)
    wp4 = jnp.zeros((TH, T), f32)
    for tt in range(T):
        wp4 = wp4.at[tt * H1:(tt + 1) * H1, tt].set(Wp[:, 0])
    w2b = jnp.zeros((TH, T * HID), f32)
    for tt in range(T):
        w2b = w2b.at[tt * H1:(tt + 1) * H1, tt * HID:(tt + 1) * HID].set(W2)

    degp = deg_k(dst_f, zeros1)
    dinv, dinvr = _tc_call(
        _tc_prep_body,
        (jax.ShapeDtypeStruct((NPAD, 1), f32),
         jax.ShapeDtypeStruct((NPAD,), f32)),
        degp)

    y14 = _tc_call(_tc1_body, jax.ShapeDtypeStruct((NPAD, TH), f32),
                   xp, W1, dinv)
    a1 = seg16(y14, src3, dst3, zeros64)
    h4, y24 = _tc_call(
        _tc2_body,
        (jax.ShapeDtypeStruct((NPAD, TH), f32),
         jax.ShapeDtypeStruct((T, NPAD), f32)),
        a1, y14, dinv, b14, wp4)
    sp = seg1(y24, src_f, dst_f, zeros4)
    keep4, hp4 = _tc_call(
        _tc3_body,
        (jax.ShapeDtypeStruct((T, NPAD), f32),
         jax.ShapeDtypeStruct((NPAD, TH), f32)),
        sp, y24, dinvr, h4, brow, bp.reshape(1, 1), c_lo, c_hi,
        smem_args=2,
        scratch_shapes=(pltpu.VMEM((T, NPAD), f32),
                        pltpu.VMEM((NPAD,), f32)))
    dp = seg1(keep4, src_f, dst_f, zeros4)
    dinv24, z4 = _tc_call(
        _tc4_body,
        (jax.ShapeDtypeStruct((T, NPAD), f32),
         jax.ShapeDtypeStruct((NPAD, TH), f32)),
        dp, keep4, hp4)
    a2 = seg16(z4, src3, dst3, zeros64)
    out = _tc_call(
        _tc5_body, jax.ShapeDtypeStruct((B, 1), f32),
        a2, z4, dinv24, keep4, w2b, b2.reshape(1, HID), bcol,
        Wih.T, Whh.T, (bih + bhh).reshape(1, 4 * HID),
        Wc.T, bc.reshape(1, 1))
    return out
